# baseline (speedup 1.0000x reference)
import functools

import jax
import jax.numpy as jnp
from jax import lax
from jax.experimental import pallas as pl
from jax.experimental.pallas import tpu as pltpu
from jax.experimental.pallas import tpu_sc as plsc

N_NODES = 10000
D_IN = 128
D_HID = 256
D_OUT = 64

NC = 2
NS = 16
NW = NC * NS

K = 128
RPW = 80
E_PAD = NW * RPW * K
NROWS = E_PAD // K

ROWS_PER_TILE = 640
NACC = NS * ROWS_PER_TILE
DEGW = 8


def _mesh():
    return plsc.VectorSubcoreMesh(
        core_axis_name="c", subcore_axis_name="s", num_cores=NC, num_subcores=NS
    )


def _sc_degree(dst2d, ones8, zeros8):
    @functools.partial(
        pl.kernel,
        out_type=jax.ShapeDtypeStruct((NC, NACC, DEGW), jnp.float32),
        mesh=_mesh(),
        scratch_types=[
            pltpu.VMEM((RPW, K), jnp.int32),
            pltpu.VMEM((K, DEGW), jnp.float32),
            pltpu.VMEM_SHARED((NACC, DEGW), jnp.float32),
        ],
        compiler_params=pltpu.CompilerParams(use_tc_tiling_on_sc=False),
    )
    def deg_kernel(dst_hbm, ones_hbm, zeros_hbm, out_hbm, dst_v, ones_v, acc):
        c = lax.axis_index("c")
        s = lax.axis_index("s")
        w = s * NC + c
        pltpu.sync_copy(zeros_hbm, acc.at[pl.ds(s * ROWS_PER_TILE, ROWS_PER_TILE)])
        pltpu.sync_copy(dst_hbm.at[pl.ds(w * RPW, RPW)], dst_v)
        pltpu.sync_copy(ones_hbm, ones_v)
        plsc.subcore_barrier()

        def body(j, carry):
            pltpu.sync_copy(ones_v, acc.at[dst_v.at[j]], add=True)
            return carry

        lax.fori_loop(0, RPW, body, 0)
        plsc.subcore_barrier()
        pltpu.sync_copy(
            acc.at[pl.ds(s * ROWS_PER_TILE, ROWS_PER_TILE)],
            out_hbm.at[c, pl.ds(s * ROWS_PER_TILE, ROWS_PER_TILE)],
        )

    return deg_kernel(dst2d, ones8, zeros8)


def _sc_prop(feat, src2d, dst2d, zerosD, d_feat):
    if d_feat > 64:
        NBUF, PPH, R0, R1 = 2, 48, 144, 16
    else:
        NBUF, PPH, R0, R1 = 4, 40, 160, 0
    C0TOT = NS * R0

    @functools.partial(
        pl.kernel,
        out_type=jax.ShapeDtypeStruct((NC, NACC, d_feat), jnp.float32),
        mesh=_mesh(),
        scratch_types=(
            [
                pltpu.VMEM((PPH, K), jnp.int32),
                pltpu.VMEM((PPH, K), jnp.int32),
            ]
            + [pltpu.VMEM((K, d_feat), jnp.float32) for _ in range(NBUF)]
            + [pltpu.VMEM_SHARED((NACC, d_feat), jnp.float32)]
            + [pltpu.SemaphoreType.DMA for _ in range(NBUF)]
        ),
        compiler_params=pltpu.CompilerParams(use_tc_tiling_on_sc=False),
    )
    def prop_kernel(feat_hbm, src_hbm, dst_hbm, zeros_hbm, out_hbm,
                    src_v, dst_v, *rest):
        bufs = rest[:NBUF]
        acc = rest[NBUF]
        gsems = rest[NBUF + 1:]
        c = lax.axis_index("c")
        s = lax.axis_index("s")
        pltpu.sync_copy(zeros_hbm, acc.at[pl.ds(s * ROWS_PER_TILE, ROWS_PER_TILE)])
        plsc.subcore_barrier()

        def run_phase(base, rows):
            pltpu.sync_copy(src_hbm.at[pl.ds(base, rows)], src_v.at[pl.ds(0, rows)])
            pltpu.sync_copy(dst_hbm.at[pl.ds(base, rows)], dst_v.at[pl.ds(0, rows)])
            for t in range(NBUF):
                pltpu.async_copy(feat_hbm.at[src_v.at[t]], bufs[t], gsems[t])

            def body(g, carry):
                for t in range(NBUF):
                    j = g * NBUF + t
                    pltpu.make_async_copy(
                        feat_hbm.at[pl.ds(0, K)], bufs[t], gsems[t]
                    ).wait()
                    pltpu.sync_copy(bufs[t], acc.at[dst_v.at[j]], add=True)

                    @pl.when(j + NBUF < rows)
                    def _():
                        pltpu.async_copy(
                            feat_hbm.at[src_v.at[j + NBUF]], bufs[t], gsems[t]
                        )

                return carry

            lax.fori_loop(0, rows // NBUF, body, 0)

        @pl.when(c == 0)
        def _():
            for p in range(R0 // PPH):
                run_phase(s * R0 + p * PPH, PPH)

        if R1 > 0:
            @pl.when(c == 1)
            def _():
                run_phase(C0TOT + s * R1, R1)

        plsc.subcore_barrier()
        pltpu.sync_copy(
            acc.at[pl.ds(s * ROWS_PER_TILE, ROWS_PER_TILE)],
            out_hbm.at[c, pl.ds(s * ROWS_PER_TILE, ROWS_PER_TILE)],
        )

    return prop_kernel(feat, src2d, dst2d, zerosD)


_BLK = 1000
_NBLK = N_NODES // _BLK


def _tc_pre(degA, degB, x):
    def body(degA_ref, degB_ref, x_ref, xs_ref, dinv_ref):
        dinv = lax.rsqrt(degA_ref[...] + degB_ref[...] + 1.0)
        dinv_ref[...] = dinv
        xs_ref[...] = x_ref[...] * dinv[:, 0:1]

    return pl.pallas_call(
        body,
        grid=(_NBLK,),
        in_specs=[
            pl.BlockSpec((_BLK, DEGW), lambda i: (i, 0)),
            pl.BlockSpec((_BLK, DEGW), lambda i: (i, 0)),
            pl.BlockSpec((_BLK, D_IN), lambda i: (i, 0)),
        ],
        out_specs=[
            pl.BlockSpec((_BLK, D_IN), lambda i: (i, 0)),
            pl.BlockSpec((_BLK, DEGW), lambda i: (i, 0)),
        ],
        out_shape=[
            jax.ShapeDtypeStruct((N_NODES, D_IN), jnp.float32),
            jax.ShapeDtypeStruct((N_NODES, DEGW), jnp.float32),
        ],
    )(degA, degB, x)


def _tc_mid(yA, yB, xs, dinv, W1, b1, W2):
    def body(yA_ref, yB_ref, xs_ref, dinv_ref, W1_ref, b1_ref, W2_ref, gs_ref):
        d = dinv_ref[...][:, 0:1]
        t = (yA_ref[...] + yB_ref[...] + xs_ref[...]) * d
        z = jnp.dot(t, W1_ref[...], preferred_element_type=jnp.float32)
        z = jnp.maximum(z + b1_ref[...], 0.0)
        g = jnp.dot(z, W2_ref[...], preferred_element_type=jnp.float32)
        gs_ref[...] = g * d

    return pl.pallas_call(
        body,
        grid=(_NBLK,),
        in_specs=[
            pl.BlockSpec((_BLK, D_IN), lambda i: (i, 0)),
            pl.BlockSpec((_BLK, D_IN), lambda i: (i, 0)),
            pl.BlockSpec((_BLK, D_IN), lambda i: (i, 0)),
            pl.BlockSpec((_BLK, DEGW), lambda i: (i, 0)),
            pl.BlockSpec((D_IN, D_HID), lambda i: (0, 0)),
            pl.BlockSpec((1, D_HID), lambda i: (0, 0)),
            pl.BlockSpec((D_HID, D_OUT), lambda i: (0, 0)),
        ],
        out_specs=pl.BlockSpec((_BLK, D_OUT), lambda i: (i, 0)),
        out_shape=jax.ShapeDtypeStruct((N_NODES, D_OUT), jnp.float32),
    )(yA, yB, xs, dinv, W1, b1, W2)


def _tc_post(uA, uB, gs, dinv, b2):
    def body(uA_ref, uB_ref, gs_ref, dinv_ref, b2_ref, out_ref):
        d = dinv_ref[...][:, 0:1]
        o = (uA_ref[...] + uB_ref[...] + gs_ref[...]) * d + b2_ref[...]
        m = jnp.max(o, axis=1, keepdims=True)
        e = o - m
        out_ref[...] = e - jnp.log(jnp.sum(jnp.exp(e), axis=1, keepdims=True))

    return pl.pallas_call(
        body,
        grid=(_NBLK,),
        in_specs=[
            pl.BlockSpec((_BLK, D_OUT), lambda i: (i, 0)),
            pl.BlockSpec((_BLK, D_OUT), lambda i: (i, 0)),
            pl.BlockSpec((_BLK, D_OUT), lambda i: (i, 0)),
            pl.BlockSpec((_BLK, DEGW), lambda i: (i, 0)),
            pl.BlockSpec((1, D_OUT), lambda i: (0, 0)),
        ],
        out_specs=pl.BlockSpec((_BLK, D_OUT), lambda i: (i, 0)),
        out_shape=jax.ShapeDtypeStruct((N_NODES, D_OUT), jnp.float32),
    )(uA, uB, gs, dinv, b2)


def kernel(x, edge_index, W1, b1, W2, b2):
    n = x.shape[0]
    src = edge_index[0].astype(jnp.int32)
    dst = edge_index[1].astype(jnp.int32)
    e = src.shape[0]
    pad = E_PAD - e
    src2d = jnp.concatenate([src, jnp.full((pad,), n, jnp.int32)]).reshape(NROWS, K)
    dst2d = jnp.concatenate([dst, jnp.full((pad,), n, jnp.int32)]).reshape(NROWS, K)

    ones8 = jnp.ones((K, DEGW), jnp.float32)
    zeros8 = jnp.zeros((ROWS_PER_TILE, DEGW), jnp.float32)
    zeros128 = jnp.zeros((ROWS_PER_TILE, D_IN), jnp.float32)
    zeros64 = jnp.zeros((ROWS_PER_TILE, D_OUT), jnp.float32)

    deg2 = _sc_degree(dst2d, ones8, zeros8)
    xs, dinv = _tc_pre(deg2[0, :n], deg2[1, :n], x)

    xs_pad = jnp.concatenate([xs, jnp.zeros((NACC - n, D_IN), jnp.float32)], axis=0)
    y2 = _sc_prop(xs_pad, src2d, dst2d, zeros128, D_IN)
    gs = _tc_mid(y2[0, :n], y2[1, :n], xs, dinv, W1, b1.reshape(1, -1), W2)

    gs_pad = jnp.concatenate([gs, jnp.zeros((NACC - n, D_OUT), jnp.float32)], axis=0)
    u2 = _sc_prop(gs_pad, src2d, dst2d, zeros64, D_OUT)
    return _tc_post(u2[0, :n], u2[1, :n], gs, dinv, b2.reshape(1, -1))


# speedup vs baseline: 1.0647x; 1.0647x over previous
import functools

import jax
import jax.numpy as jnp
from jax import lax
from jax.experimental import pallas as pl
from jax.experimental.pallas import tpu as pltpu
from jax.experimental.pallas import tpu_sc as plsc

N_NODES = 10000
D_IN = 128
D_HID = 256
D_OUT = 64

NC = 2
NS = 16
NW = NC * NS

K = 128
RPW = 80
E_PAD = NW * RPW * K
NROWS = E_PAD // K

ROWS_PER_TILE = 640
NACC = NS * ROWS_PER_TILE
DEGW = 8


def _mesh():
    return plsc.VectorSubcoreMesh(
        core_axis_name="c", subcore_axis_name="s", num_cores=NC, num_subcores=NS
    )


def _sc_degree(dst2d, ones8, zeros8):
    @functools.partial(
        pl.kernel,
        out_type=jax.ShapeDtypeStruct((NC, NACC, DEGW), jnp.float32),
        mesh=_mesh(),
        scratch_types=[
            pltpu.VMEM((RPW, K), jnp.int32),
            pltpu.VMEM((K, DEGW), jnp.float32),
            pltpu.VMEM_SHARED((NACC, DEGW), jnp.float32),
        ],
        compiler_params=pltpu.CompilerParams(use_tc_tiling_on_sc=False),
    )
    def deg_kernel(dst_hbm, ones_hbm, zeros_hbm, out_hbm, dst_v, ones_v, acc):
        c = lax.axis_index("c")
        s = lax.axis_index("s")
        w = s * NC + c
        pltpu.sync_copy(zeros_hbm, acc.at[pl.ds(s * ROWS_PER_TILE, ROWS_PER_TILE)])
        pltpu.sync_copy(dst_hbm.at[pl.ds(w * RPW, RPW)], dst_v)
        pltpu.sync_copy(ones_hbm, ones_v)
        plsc.subcore_barrier()

        def body(j, carry):
            pltpu.sync_copy(ones_v, acc.at[dst_v.at[j]], add=True)
            return carry

        lax.fori_loop(0, RPW, body, 0)
        plsc.subcore_barrier()
        pltpu.sync_copy(
            acc.at[pl.ds(s * ROWS_PER_TILE, ROWS_PER_TILE)],
            out_hbm.at[c, pl.ds(s * ROWS_PER_TILE, ROWS_PER_TILE)],
        )

    return deg_kernel(dst2d, ones8, zeros8)


def _sc_prop(feat, src2d, dst2d, zerosD, d_feat):
    if d_feat > 64:
        NBUF, PPH, R0, R1 = 2, 48, 144, 16
    else:
        NBUF, PPH, R0, R1 = 4, 48, 144, 16
    C0TOT = NS * R0

    @functools.partial(
        pl.kernel,
        out_type=jax.ShapeDtypeStruct((NC, NACC, d_feat), jnp.float32),
        mesh=_mesh(),
        scratch_types=(
            [
                pltpu.VMEM((PPH, K), jnp.int32),
                pltpu.VMEM((PPH, K), jnp.int32),
            ]
            + [pltpu.VMEM((K, d_feat), jnp.float32) for _ in range(NBUF)]
            + [pltpu.VMEM_SHARED((NACC, d_feat), jnp.float32)]
            + [pltpu.SemaphoreType.DMA for _ in range(NBUF)]
        ),
        compiler_params=pltpu.CompilerParams(use_tc_tiling_on_sc=False),
    )
    def prop_kernel(feat_hbm, src_hbm, dst_hbm, zeros_hbm, out_hbm,
                    src_v, dst_v, *rest):
        bufs = rest[:NBUF]
        acc = rest[NBUF]
        gsems = rest[NBUF + 1:]
        c = lax.axis_index("c")
        s = lax.axis_index("s")
        pltpu.sync_copy(zeros_hbm, acc.at[pl.ds(s * ROWS_PER_TILE, ROWS_PER_TILE)])
        plsc.subcore_barrier()

        def run_phase(base, rows):
            pltpu.sync_copy(src_hbm.at[pl.ds(base, rows)], src_v.at[pl.ds(0, rows)])
            pltpu.sync_copy(dst_hbm.at[pl.ds(base, rows)], dst_v.at[pl.ds(0, rows)])
            for t in range(NBUF):
                pltpu.async_copy(feat_hbm.at[src_v.at[t]], bufs[t], gsems[t])

            def body(g, carry):
                for t in range(NBUF):
                    j = g * NBUF + t
                    pltpu.make_async_copy(
                        feat_hbm.at[pl.ds(0, K)], bufs[t], gsems[t]
                    ).wait()
                    pltpu.sync_copy(bufs[t], acc.at[dst_v.at[j]], add=True)

                    @pl.when(j + NBUF < rows)
                    def _():
                        pltpu.async_copy(
                            feat_hbm.at[src_v.at[j + NBUF]], bufs[t], gsems[t]
                        )

                return carry

            lax.fori_loop(0, rows // NBUF, body, 0)

        @pl.when(c == 0)
        def _():
            for p in range(R0 // PPH):
                run_phase(s * R0 + p * PPH, PPH)

        if R1 > 0:
            @pl.when(c == 1)
            def _():
                run_phase(C0TOT + s * R1, R1)

        plsc.subcore_barrier()
        pltpu.sync_copy(
            acc.at[pl.ds(s * ROWS_PER_TILE, ROWS_PER_TILE)],
            out_hbm.at[c, pl.ds(s * ROWS_PER_TILE, ROWS_PER_TILE)],
        )

    return prop_kernel(feat, src2d, dst2d, zerosD)


_BLK = 1000
_NBLK = N_NODES // _BLK


def _tc_pre(degA, degB, x):
    def body(degA_ref, degB_ref, x_ref, xs_ref, dinv_ref):
        dinv = lax.rsqrt(degA_ref[...] + degB_ref[...] + 1.0)
        dinv_ref[...] = dinv
        xs_ref[...] = x_ref[...] * dinv[:, 0:1]

    return pl.pallas_call(
        body,
        grid=(_NBLK,),
        in_specs=[
            pl.BlockSpec((_BLK, DEGW), lambda i: (i, 0)),
            pl.BlockSpec((_BLK, DEGW), lambda i: (i, 0)),
            pl.BlockSpec((_BLK, D_IN), lambda i: (i, 0)),
        ],
        out_specs=[
            pl.BlockSpec((_BLK, D_IN), lambda i: (i, 0)),
            pl.BlockSpec((_BLK, DEGW), lambda i: (i, 0)),
        ],
        out_shape=[
            jax.ShapeDtypeStruct((N_NODES, D_IN), jnp.float32),
            jax.ShapeDtypeStruct((N_NODES, DEGW), jnp.float32),
        ],
    )(degA, degB, x)


def _tc_mid(yA, yB, xs, dinv, W1, b1, W2):
    def body(yA_ref, yB_ref, xs_ref, dinv_ref, W1_ref, b1_ref, W2_ref, gs_ref):
        d = dinv_ref[...][:, 0:1]
        t = (yA_ref[...] + yB_ref[...] + xs_ref[...]) * d
        z = jnp.dot(t, W1_ref[...], preferred_element_type=jnp.float32)
        z = jnp.maximum(z + b1_ref[...], 0.0)
        g = jnp.dot(z, W2_ref[...], preferred_element_type=jnp.float32)
        gs_ref[...] = g * d

    return pl.pallas_call(
        body,
        grid=(_NBLK,),
        in_specs=[
            pl.BlockSpec((_BLK, D_IN), lambda i: (i, 0)),
            pl.BlockSpec((_BLK, D_IN), lambda i: (i, 0)),
            pl.BlockSpec((_BLK, D_IN), lambda i: (i, 0)),
            pl.BlockSpec((_BLK, DEGW), lambda i: (i, 0)),
            pl.BlockSpec((D_IN, D_HID), lambda i: (0, 0)),
            pl.BlockSpec((1, D_HID), lambda i: (0, 0)),
            pl.BlockSpec((D_HID, D_OUT), lambda i: (0, 0)),
        ],
        out_specs=pl.BlockSpec((_BLK, D_OUT), lambda i: (i, 0)),
        out_shape=jax.ShapeDtypeStruct((N_NODES, D_OUT), jnp.float32),
    )(yA, yB, xs, dinv, W1, b1, W2)


def _tc_post(uA, uB, gs, dinv, b2):
    def body(uA_ref, uB_ref, gs_ref, dinv_ref, b2_ref, out_ref):
        d = dinv_ref[...][:, 0:1]
        o = (uA_ref[...] + uB_ref[...] + gs_ref[...]) * d + b2_ref[...]
        m = jnp.max(o, axis=1, keepdims=True)
        e = o - m
        out_ref[...] = e - jnp.log(jnp.sum(jnp.exp(e), axis=1, keepdims=True))

    return pl.pallas_call(
        body,
        grid=(_NBLK,),
        in_specs=[
            pl.BlockSpec((_BLK, D_OUT), lambda i: (i, 0)),
            pl.BlockSpec((_BLK, D_OUT), lambda i: (i, 0)),
            pl.BlockSpec((_BLK, D_OUT), lambda i: (i, 0)),
            pl.BlockSpec((_BLK, DEGW), lambda i: (i, 0)),
            pl.BlockSpec((1, D_OUT), lambda i: (0, 0)),
        ],
        out_specs=pl.BlockSpec((_BLK, D_OUT), lambda i: (i, 0)),
        out_shape=jax.ShapeDtypeStruct((N_NODES, D_OUT), jnp.float32),
    )(uA, uB, gs, dinv, b2)


def kernel(x, edge_index, W1, b1, W2, b2):
    n = x.shape[0]
    src = edge_index[0].astype(jnp.int32)
    dst = edge_index[1].astype(jnp.int32)
    e = src.shape[0]
    pad = E_PAD - e
    src2d = jnp.concatenate([src, jnp.full((pad,), n, jnp.int32)]).reshape(NROWS, K)
    dst2d = jnp.concatenate([dst, jnp.full((pad,), n, jnp.int32)]).reshape(NROWS, K)

    ones8 = jnp.ones((K, DEGW), jnp.float32)
    zeros8 = jnp.zeros((ROWS_PER_TILE, DEGW), jnp.float32)
    zeros128 = jnp.zeros((ROWS_PER_TILE, D_IN), jnp.float32)
    zeros64 = jnp.zeros((ROWS_PER_TILE, D_OUT), jnp.float32)

    deg2 = _sc_degree(dst2d, ones8, zeros8)
    xs, dinv = _tc_pre(deg2[0, :n], deg2[1, :n], x)

    xs_pad = jnp.concatenate([xs, jnp.zeros((NACC - n, D_IN), jnp.float32)], axis=0)
    y2 = _sc_prop(xs_pad, src2d, dst2d, zeros128, D_IN)
    gs = _tc_mid(y2[0, :n], y2[1, :n], xs, dinv, W1, b1.reshape(1, -1), W2)

    gs_pad = jnp.concatenate([gs, jnp.zeros((NACC - n, D_OUT), jnp.float32)], axis=0)
    u2 = _sc_prop(gs_pad, src2d, dst2d, zeros64, D_OUT)
    return _tc_post(u2[0, :n], u2[1, :n], gs, dinv, b2.reshape(1, -1))
